# Initial kernel scaffold; baseline (speedup 1.0000x reference)
import jax
import jax.numpy as jnp
from jax.experimental import pallas as pl


def kernel(x_0, x_1, x_2, adjacency_0, adjacency_1, coadjacency_2, incidence_1_rows, incidence_1_cols, incidence_2_rows, incidence_2_cols, W_in0, b_in0, W_in1, b_in1, W_in2, b_in2, W00, W10, W01, W11, W12, W22, V00, V01, V11, V12, V22, Wo0, bo0, Wo1, bo1, Wo2, bo2):
    raise NotImplementedError("write your pallas kernel here")


# speedup vs baseline: 2.8944x; 2.8944x over previous
import functools

import jax
import jax.numpy as jnp
from jax import lax
from jax.experimental import pallas as pl
from jax.experimental.pallas import tpu as pltpu
from jax.experimental.pallas import tpu_sc as plsc

N0 = 50000
N1 = 100000
N2 = 50000
HP = 32
HH = 16
CH = 128
NSUB = 16
EGRAN = NSUB * CH
ACC_ROWS = 102400
GROW = ACC_ROWS - 1
ZB = 1024


def _pad_mat(w):
    r, c = w.shape
    return jnp.zeros((HP, HP), jnp.float32).at[:r, :c].set(w)


def _pad_vec_mat(b, w):
    v = b @ w
    return jnp.zeros((HP,), jnp.float32).at[: v.shape[0]].set(v)


def _prep_src(idx):
    e = idx.shape[0]
    ep = -(-e // EGRAN) * EGRAN
    return jnp.concatenate(
        [idx.astype(jnp.int32), jnp.zeros((ep - e,), jnp.int32)])


def _prep_dst(idx):
    e = idx.shape[0]
    ep = -(-e // EGRAN) * EGRAN
    return jnp.concatenate(
        [idx.astype(jnp.int32), jnp.full((ep - e,), GROW, jnp.int32)])


def _tc_transform(x, t_mat, t_bias):
    n, k = x.shape[0], t_mat.shape[1]
    bn = 400
    no = k // HH

    def body(x_ref, tm_ref, tb_ref, *outs):
        y = jnp.dot(x_ref[...], tm_ref[...],
                    preferred_element_type=jnp.float32) + tb_ref[...]
        for j, o in enumerate(outs):
            o[...] = y[:, j * HH:(j + 1) * HH]

    return pl.pallas_call(
        body,
        grid=(n // bn,),
        in_specs=[pl.BlockSpec((bn, HP), lambda i: (i, 0)),
                  pl.BlockSpec((HP, k), lambda i: (0, 0)),
                  pl.BlockSpec((1, k), lambda i: (0, 0))],
        out_specs=[pl.BlockSpec((bn, HH), lambda i: (i, 0))] * no,
        out_shape=[jax.ShapeDtypeStruct((n, HH), jnp.float32)] * no,
    )(x, t_mat, t_bias)


def _tc_transform2(a, b, u_mat):
    n, k = a.shape[0], u_mat.shape[1]
    bn = 400
    no = k // HH

    def body(a_ref, b_ref, um_ref, *outs):
        h = jax.nn.relu(jnp.concatenate([a_ref[...], b_ref[...]], axis=1))
        y = jnp.dot(h, um_ref[...], preferred_element_type=jnp.float32)
        for j, o in enumerate(outs):
            o[...] = y[:, j * HH:(j + 1) * HH]

    return pl.pallas_call(
        body,
        grid=(n // bn,),
        in_specs=[pl.BlockSpec((bn, HH), lambda i: (i, 0)),
                  pl.BlockSpec((bn, HH), lambda i: (i, 0)),
                  pl.BlockSpec((HP, k), lambda i: (0, 0))],
        out_specs=[pl.BlockSpec((bn, HH), lambda i: (i, 0))] * no,
        out_shape=[jax.ShapeDtypeStruct((n, HH), jnp.float32)] * no,
    )(a, b, u_mat)


def _tc_reduce(a, b):
    n = a.shape[0]
    bn = 2000

    def body(a_ref, b_ref, o_ref):
        @pl.when(pl.program_id(0) == 0)
        def _():
            o_ref[...] = jnp.zeros_like(o_ref)

        h = jax.nn.relu(jnp.concatenate([a_ref[...], b_ref[...]], axis=1))
        o_ref[...] += jnp.sum(h, axis=0, keepdims=True)

    return pl.pallas_call(
        body,
        grid=(n // bn,),
        in_specs=[pl.BlockSpec((bn, HH), lambda i: (i, 0)),
                  pl.BlockSpec((bn, HH), lambda i: (i, 0))],
        out_specs=pl.BlockSpec((1, HP), lambda i: (0, 0)),
        out_shape=jax.ShapeDtypeStruct((1, HP), jnp.float32),
    )(a, b)


def _sc_layer(stages):
    ins = []
    meta = []
    for n_rows, jobs in stages:
        jmeta = []
        for (src, dst, t0, t1) in jobs:
            jmeta.append((len(ins), src.shape[0]))
            ins.extend([src, dst, t0, t1])
        meta.append((n_rows, jmeta))

    out_type = []
    for n_rows, _ in stages:
        out_type.extend([jax.ShapeDtypeStruct((n_rows, HH), jnp.float32)] * 2)

    mesh = plsc.VectorSubcoreMesh(core_axis_name="core",
                                  subcore_axis_name="subcore")

    @functools.partial(
        pl.kernel,
        out_type=out_type,
        mesh=mesh,
        compiler_params=pltpu.CompilerParams(use_tc_tiling_on_sc=False),
        scratch_types=[
            pltpu.VMEM_SHARED((ACC_ROWS, HH), jnp.float32),
            pltpu.VMEM((ZB, HH), jnp.float32),
            pltpu.VMEM((1, CH), jnp.int32),
            pltpu.VMEM((1, CH), jnp.int32),
            pltpu.VMEM((CH, HH), jnp.float32),
        ],
    )
    def sc_kernel(*refs):
        n_in = len(ins)
        in_refs = refs[:n_in]
        out_refs = refs[n_in:n_in + 2 * len(stages)]
        acc, zbuf, srcb, dstb, rowb = refs[n_in + 2 * len(stages):]

        cid = lax.axis_index("core")
        sid = lax.axis_index("subcore")

        @pl.loop(0, ZB)
        def _(i):
            zbuf[i, :] = jnp.zeros((HH,), jnp.float32)

        for si, (n_rows, jmeta) in enumerate(meta):
            per = -(-(n_rows // NSUB) // 8) * 8
            last = n_rows - (NSUB - 1) * per
            rbase = sid * per
            def zero_slice(m, rbase=rbase):
                off = 0
                while off < m:
                    step = min(ZB, m - off)
                    pltpu.sync_copy(zbuf.at[pl.ds(0, step)],
                                    acc.at[pl.ds(rbase + off, step)])
                    off += step

            @pl.when(sid < NSUB - 1)
            def _():
                zero_slice(per)

            @pl.when(sid == NSUB - 1)
            def _():
                zero_slice(last)

            plsc.subcore_barrier()

            for (base_i, e_pad) in jmeta:
                src_ref = in_refs[base_i]
                dst_ref = in_refs[base_i + 1]
                nch = e_pad // (NSUB * CH)
                ebase = sid * (e_pad // NSUB)

                def run_job(table_ref, nch=nch, ebase=ebase,
                            src_ref=src_ref, dst_ref=dst_ref):
                    @pl.loop(0, nch)
                    def _(i):
                        o = ebase + i * CH
                        pltpu.sync_copy(src_ref.at[pl.ds(o, CH)], srcb.at[0])
                        pltpu.sync_copy(dst_ref.at[pl.ds(o, CH)], dstb.at[0])
                        pltpu.sync_copy(table_ref.at[srcb.at[0]], rowb)
                        pltpu.sync_copy(rowb, acc.at[dstb.at[0]], add=True)

                @pl.when(cid == 0)
                def _():
                    run_job(in_refs[base_i + 2])

                @pl.when(cid == 1)
                def _():
                    run_job(in_refs[base_i + 3])

            plsc.subcore_barrier()

            def drain(out_ref, rbase=rbase, per=per, last=last):
                @pl.when(sid < NSUB - 1)
                def _():
                    pltpu.sync_copy(acc.at[pl.ds(rbase, per)],
                                    out_ref.at[pl.ds(rbase, per)])

                @pl.when(sid == NSUB - 1)
                def _():
                    pltpu.sync_copy(acc.at[pl.ds(rbase, last)],
                                    out_ref.at[pl.ds(rbase, last)])

            @pl.when(cid == 0)
            def _():
                drain(out_refs[2 * si])

            @pl.when(cid == 1)
            def _():
                drain(out_refs[2 * si + 1])

            plsc.subcore_barrier()

    return sc_kernel(*ins)


def kernel(x_0, x_1, x_2, adjacency_0, adjacency_1, coadjacency_2,
           incidence_1_rows, incidence_1_cols, incidence_2_rows,
           incidence_2_cols,
           W_in0, b_in0, W_in1, b_in1, W_in2, b_in2,
           W00, W10, W01, W11, W12, W22,
           V00, V01, V11, V12, V22,
           Wo0, bo0, Wo1, bo1, Wo2, bo2):
    t0_mat = jnp.concatenate(
        [_pad_mat(W_in0 @ W00), _pad_mat(W_in0 @ W01)], axis=1)
    t0_bias = jnp.concatenate(
        [_pad_vec_mat(b_in0, W00), _pad_vec_mat(b_in0, W01)])[None, :]
    t1_mat = jnp.concatenate(
        [_pad_mat(W_in1 @ W10), _pad_mat(W_in1 @ W11),
         _pad_mat(W_in1 @ W12)], axis=1)
    t1_bias = jnp.concatenate(
        [_pad_vec_mat(b_in1, W10), _pad_vec_mat(b_in1, W11),
         _pad_vec_mat(b_in1, W12)])[None, :]
    t2_mat = _pad_mat(W_in2 @ W22)
    t2_bias = _pad_vec_mat(b_in2, W22)[None, :]

    u0_mat = jnp.concatenate([_pad_mat(V00), _pad_mat(V01)], axis=1)
    u1_mat = jnp.concatenate([_pad_mat(V11), _pad_mat(V12)], axis=1)
    u2_mat = _pad_mat(V22)

    a0_dst, a0_src = _prep_dst(adjacency_0[0]), _prep_src(adjacency_0[1])
    a1_dst, a1_src = _prep_dst(adjacency_1[0]), _prep_src(adjacency_1[1])
    c2_dst, c2_src = _prep_dst(coadjacency_2[0]), _prep_src(coadjacency_2[1])
    i1r_src = _prep_src(incidence_1_rows)
    i1r_dst = _prep_dst(incidence_1_rows)
    i1c_src = _prep_src(incidence_1_cols)
    i1c_dst = _prep_dst(incidence_1_cols)
    i2r_src = _prep_src(incidence_2_rows)
    i2c_dst = _prep_dst(incidence_2_cols)

    g00_0, g00_1, g01_0, g01_1 = _tc_transform(x_0, t0_mat, t0_bias)
    g10_0, g10_1, g11_0, g11_1, g12_0, g12_1 = _tc_transform(
        x_1, t1_mat, t1_bias)
    g22_0, g22_1 = _tc_transform(x_2, t2_mat, t2_bias)

    (acc0_0, acc0_1, acc1_0, acc1_1, acc2_0, acc2_1) = _sc_layer([
        (N0, [(a0_src, a0_dst, g00_0, g00_1),
              (i1c_src, i1r_dst, g10_0, g10_1)]),
        (N1, [(i1r_src, i1c_dst, g01_0, g01_1),
              (a1_src, a1_dst, g11_0, g11_1)]),
        (N2, [(i2r_src, i2c_dst, g12_0, g12_1),
              (c2_src, c2_dst, g22_0, g22_1)]),
    ])

    d00_0, d00_1, d01_0, d01_1 = _tc_transform2(acc0_0, acc0_1, u0_mat)
    d11_0, d11_1, d12_0, d12_1 = _tc_transform2(acc1_0, acc1_1, u1_mat)
    d22_0, d22_1 = _tc_transform2(acc2_0, acc2_1, u2_mat)

    (b0_0, b0_1, b1_0, b1_1, b2_0, b2_1) = _sc_layer([
        (N0, [(a0_src, a0_dst, d00_0, d00_1)]),
        (N1, [(i1r_src, i1c_dst, d01_0, d01_1),
              (a1_src, a1_dst, d11_0, d11_1)]),
        (N2, [(i2r_src, i2c_dst, d12_0, d12_1),
              (c2_src, c2_dst, d22_0, d22_1)]),
    ])

    s0 = _tc_reduce(b0_0, b0_1)
    s1 = _tc_reduce(b1_0, b1_1)
    s2 = _tc_reduce(b2_0, b2_1)

    def head(s, n, wo, bo):
        wo_p = jnp.zeros((HP, 1), jnp.float32).at[:wo.shape[0]].set(wo)
        return (s @ wo_p)[0] / n + bo

    return (head(s0, N0, Wo0, bo0) + head(s1, N1, Wo1, bo1)
            + head(s2, N2, Wo2, bo2))


# speedup vs baseline: 7.0292x; 2.4286x over previous
import functools

import jax
import jax.numpy as jnp
from jax import lax
from jax.experimental import pallas as pl
from jax.experimental.pallas import tpu as pltpu
from jax.experimental.pallas import tpu_sc as plsc

N0 = 50000
N1 = 100000
N2 = 50000
HP = 32
HH = 16
CH = 128
GR = 8
NSUB = 16
EGRAN = NSUB * CH * GR
ACC_ROWS = 102400
GROW = ACC_ROWS - 1
ZB = 256


def _pad_mat(w):
    r, c = w.shape
    return jnp.zeros((HP, HP), jnp.float32).at[:r, :c].set(w)


def _pad_vec_mat(b, w):
    v = b @ w
    return jnp.zeros((HP,), jnp.float32).at[: v.shape[0]].set(v)


def _prep_src(idx):
    e = idx.shape[0]
    ep = -(-e // EGRAN) * EGRAN
    return jnp.concatenate(
        [idx.astype(jnp.int32),
         jnp.zeros((ep - e,), jnp.int32)]).reshape(ep // CH, CH)


def _prep_dst(idx):
    e = idx.shape[0]
    ep = -(-e // EGRAN) * EGRAN
    return jnp.concatenate(
        [idx.astype(jnp.int32),
         jnp.full((ep - e,), GROW, jnp.int32)]).reshape(ep // CH, CH)


def _tc_transform(x, t_mat, t_bias):
    n, k = x.shape[0], t_mat.shape[1]
    bn = 400
    no = k // HH

    def body(x_ref, tm_ref, tb_ref, *outs):
        y = jnp.dot(x_ref[...], tm_ref[...],
                    preferred_element_type=jnp.float32) + tb_ref[...]
        for j, o in enumerate(outs):
            o[...] = y[:, j * HH:(j + 1) * HH]

    return pl.pallas_call(
        body,
        grid=(n // bn,),
        in_specs=[pl.BlockSpec((bn, HP), lambda i: (i, 0)),
                  pl.BlockSpec((HP, k), lambda i: (0, 0)),
                  pl.BlockSpec((1, k), lambda i: (0, 0))],
        out_specs=[pl.BlockSpec((bn, HH), lambda i: (i, 0))] * no,
        out_shape=[jax.ShapeDtypeStruct((n, HH), jnp.float32)] * no,
    )(x, t_mat, t_bias)


def _tc_transform2(a, b, u_mat):
    n, k = a.shape[0], u_mat.shape[1]
    bn = 400
    no = k // HH

    def body(a_ref, b_ref, um_ref, *outs):
        h = jax.nn.relu(jnp.concatenate([a_ref[...], b_ref[...]], axis=1))
        y = jnp.dot(h, um_ref[...], preferred_element_type=jnp.float32)
        for j, o in enumerate(outs):
            o[...] = y[:, j * HH:(j + 1) * HH]

    return pl.pallas_call(
        body,
        grid=(n // bn,),
        in_specs=[pl.BlockSpec((bn, HH), lambda i: (i, 0)),
                  pl.BlockSpec((bn, HH), lambda i: (i, 0)),
                  pl.BlockSpec((HP, k), lambda i: (0, 0))],
        out_specs=[pl.BlockSpec((bn, HH), lambda i: (i, 0))] * no,
        out_shape=[jax.ShapeDtypeStruct((n, HH), jnp.float32)] * no,
    )(a, b, u_mat)


def _tc_reduce(a, b):
    n = a.shape[0]
    bn = 2000

    def body(a_ref, b_ref, o_ref):
        @pl.when(pl.program_id(0) == 0)
        def _():
            o_ref[...] = jnp.zeros_like(o_ref)

        h = jax.nn.relu(jnp.concatenate([a_ref[...], b_ref[...]], axis=1))
        o_ref[...] += jnp.sum(h, axis=0, keepdims=True)

    return pl.pallas_call(
        body,
        grid=(n // bn,),
        in_specs=[pl.BlockSpec((bn, HH), lambda i: (i, 0)),
                  pl.BlockSpec((bn, HH), lambda i: (i, 0))],
        out_specs=pl.BlockSpec((1, HP), lambda i: (0, 0)),
        out_shape=jax.ShapeDtypeStruct((1, HP), jnp.float32),
    )(a, b)


def _sc_layer(stages):
    ins = []
    meta = []
    for n_rows, jobs in stages:
        jmeta = []
        for (src, dst, t0, t1) in jobs:
            jmeta.append((len(ins), src.shape[0]))
            ins.extend([src, dst, t0, t1])
        meta.append((n_rows, jmeta))

    out_type = []
    for n_rows, _ in stages:
        out_type.extend([jax.ShapeDtypeStruct((n_rows, HH), jnp.float32)] * 2)

    mesh = plsc.VectorSubcoreMesh(core_axis_name="core",
                                  subcore_axis_name="subcore")

    @functools.partial(
        pl.kernel,
        out_type=out_type,
        mesh=mesh,
        compiler_params=pltpu.CompilerParams(use_tc_tiling_on_sc=False),
        scratch_types=[
            pltpu.VMEM_SHARED((ACC_ROWS, HH), jnp.float32),
            pltpu.VMEM((ZB, HH), jnp.float32),
            pltpu.VMEM((2, GR, CH), jnp.int32),
            pltpu.VMEM((2, GR, CH), jnp.int32),
            pltpu.VMEM((GR, CH, HH), jnp.float32),
        ] + [pltpu.SemaphoreType.DMA] * (GR + 3),
    )
    def sc_kernel(*refs):
        n_in = len(ins)
        in_refs = refs[:n_in]
        out_refs = refs[n_in:n_in + 2 * len(stages)]
        scr = refs[n_in + 2 * len(stages):]
        acc, zbuf, srcb, dstb, rowb = scr[:5]
        gsem = scr[5:5 + GR]
        ssem = scr[5 + GR]
        isem = scr[6 + GR:8 + GR]

        cid = lax.axis_index("core")
        sid = lax.axis_index("subcore")

        @pl.loop(0, ZB)
        def _(i):
            zbuf[i, :] = jnp.zeros((HH,), jnp.float32)

        for si, (n_rows, jmeta) in enumerate(meta):
            per = -(-(n_rows // NSUB) // 8) * 8
            last = n_rows - (NSUB - 1) * per
            rbase = sid * per
            def zero_slice(m, rbase=rbase):
                off = 0
                while off < m:
                    step = min(ZB, m - off)
                    pltpu.sync_copy(zbuf.at[pl.ds(0, step)],
                                    acc.at[pl.ds(rbase + off, step)])
                    off += step

            @pl.when(sid < NSUB - 1)
            def _():
                zero_slice(per)

            @pl.when(sid == NSUB - 1)
            def _():
                zero_slice(last)

            plsc.subcore_barrier()

            for (base_i, idx_rows) in jmeta:
                src_ref = in_refs[base_i]
                dst_ref = in_refs[base_i + 1]
                nch = idx_rows // NSUB
                ngr = nch // GR
                crow0 = sid * nch

                def run_job(table_ref, ngr=ngr, crow0=crow0,
                            src_ref=src_ref, dst_ref=dst_ref):
                    def load_idx(g, b):
                        r = crow0 + g * GR
                        pltpu.async_copy(src_ref.at[pl.ds(r, GR)],
                                         srcb.at[b], isem[b])
                        pltpu.async_copy(dst_ref.at[pl.ds(r, GR)],
                                         dstb.at[b], isem[b])

                    def wait_idx(g, b):
                        r = crow0 + g * GR
                        pltpu.make_async_copy(src_ref.at[pl.ds(r, GR)],
                                              srcb.at[b], isem[b]).wait()
                        pltpu.make_async_copy(dst_ref.at[pl.ds(r, GR)],
                                              dstb.at[b], isem[b]).wait()

                    def do_group(g, b):
                        wait_idx(g, b)

                        @pl.when(g + 1 < ngr)
                        def _():
                            load_idx(g + 1, 1 - b)

                        gcps = [pltpu.async_copy(
                                    table_ref.at[srcb.at[b, k]],
                                    rowb.at[k], gsem[k])
                                for k in range(GR)]
                        scps = []
                        for k in range(GR):
                            gcps[k].wait()
                            scps.append(pltpu.async_copy(
                                rowb.at[k], acc.at[dstb.at[b, k]],
                                ssem, add=True))
                        for cp in scps:
                            cp.wait()

                    load_idx(0, 0)
                    if ngr >= 2:
                        @pl.loop(0, ngr // 2)
                        def _(t):
                            do_group(2 * t, 0)
                            do_group(2 * t + 1, 1)
                    if ngr % 2:
                        do_group(ngr - 1, 0)

                @pl.when(cid == 0)
                def _():
                    run_job(in_refs[base_i + 2])

                @pl.when(cid == 1)
                def _():
                    run_job(in_refs[base_i + 3])

            plsc.subcore_barrier()

            def drain(out_ref, rbase=rbase, per=per, last=last):
                @pl.when(sid < NSUB - 1)
                def _():
                    pltpu.sync_copy(acc.at[pl.ds(rbase, per)],
                                    out_ref.at[pl.ds(rbase, per)])

                @pl.when(sid == NSUB - 1)
                def _():
                    pltpu.sync_copy(acc.at[pl.ds(rbase, last)],
                                    out_ref.at[pl.ds(rbase, last)])

            @pl.when(cid == 0)
            def _():
                drain(out_refs[2 * si])

            @pl.when(cid == 1)
            def _():
                drain(out_refs[2 * si + 1])

            plsc.subcore_barrier()

    return sc_kernel(*ins)


def kernel(x_0, x_1, x_2, adjacency_0, adjacency_1, coadjacency_2,
           incidence_1_rows, incidence_1_cols, incidence_2_rows,
           incidence_2_cols,
           W_in0, b_in0, W_in1, b_in1, W_in2, b_in2,
           W00, W10, W01, W11, W12, W22,
           V00, V01, V11, V12, V22,
           Wo0, bo0, Wo1, bo1, Wo2, bo2):
    t0_mat = jnp.concatenate(
        [_pad_mat(W_in0 @ W00), _pad_mat(W_in0 @ W01)], axis=1)
    t0_bias = jnp.concatenate(
        [_pad_vec_mat(b_in0, W00), _pad_vec_mat(b_in0, W01)])[None, :]
    t1_mat = jnp.concatenate(
        [_pad_mat(W_in1 @ W10), _pad_mat(W_in1 @ W11),
         _pad_mat(W_in1 @ W12)], axis=1)
    t1_bias = jnp.concatenate(
        [_pad_vec_mat(b_in1, W10), _pad_vec_mat(b_in1, W11),
         _pad_vec_mat(b_in1, W12)])[None, :]
    t2_mat = _pad_mat(W_in2 @ W22)
    t2_bias = _pad_vec_mat(b_in2, W22)[None, :]

    u0_mat = jnp.concatenate([_pad_mat(V00), _pad_mat(V01)], axis=1)
    u1_mat = jnp.concatenate([_pad_mat(V11), _pad_mat(V12)], axis=1)
    u2_mat = _pad_mat(V22)

    a0_dst, a0_src = _prep_dst(adjacency_0[0]), _prep_src(adjacency_0[1])
    a1_dst, a1_src = _prep_dst(adjacency_1[0]), _prep_src(adjacency_1[1])
    c2_dst, c2_src = _prep_dst(coadjacency_2[0]), _prep_src(coadjacency_2[1])
    i1r_src = _prep_src(incidence_1_rows)
    i1r_dst = _prep_dst(incidence_1_rows)
    i1c_src = _prep_src(incidence_1_cols)
    i1c_dst = _prep_dst(incidence_1_cols)
    i2r_src = _prep_src(incidence_2_rows)
    i2c_dst = _prep_dst(incidence_2_cols)

    g00_0, g00_1, g01_0, g01_1 = _tc_transform(x_0, t0_mat, t0_bias)
    g10_0, g10_1, g11_0, g11_1, g12_0, g12_1 = _tc_transform(
        x_1, t1_mat, t1_bias)
    g22_0, g22_1 = _tc_transform(x_2, t2_mat, t2_bias)

    (acc0_0, acc0_1, acc1_0, acc1_1, acc2_0, acc2_1) = _sc_layer([
        (N0, [(a0_src, a0_dst, g00_0, g00_1),
              (i1c_src, i1r_dst, g10_0, g10_1)]),
        (N1, [(i1r_src, i1c_dst, g01_0, g01_1),
              (a1_src, a1_dst, g11_0, g11_1)]),
        (N2, [(i2r_src, i2c_dst, g12_0, g12_1),
              (c2_src, c2_dst, g22_0, g22_1)]),
    ])

    d00_0, d00_1, d01_0, d01_1 = _tc_transform2(acc0_0, acc0_1, u0_mat)
    d11_0, d11_1, d12_0, d12_1 = _tc_transform2(acc1_0, acc1_1, u1_mat)
    d22_0, d22_1 = _tc_transform2(acc2_0, acc2_1, u2_mat)

    (b0_0, b0_1, b1_0, b1_1, b2_0, b2_1) = _sc_layer([
        (N0, [(a0_src, a0_dst, d00_0, d00_1)]),
        (N1, [(i1r_src, i1c_dst, d01_0, d01_1),
              (a1_src, a1_dst, d11_0, d11_1)]),
        (N2, [(i2r_src, i2c_dst, d12_0, d12_1),
              (c2_src, c2_dst, d22_0, d22_1)]),
    ])

    s0 = _tc_reduce(b0_0, b0_1)
    s1 = _tc_reduce(b1_0, b1_1)
    s2 = _tc_reduce(b2_0, b2_1)

    def head(s, n, wo, bo):
        wo_p = jnp.zeros((HP, 1), jnp.float32).at[:wo.shape[0]].set(wo)
        return (s @ wo_p)[0] / n + bo

    return (head(s0, N0, Wo0, bo0) + head(s1, N1, Wo1, bo1)
            + head(s2, N2, Wo2, bo2))


# speedup vs baseline: 7.9349x; 1.1288x over previous
import functools

import jax
import jax.numpy as jnp
from jax import lax
from jax.experimental import pallas as pl
from jax.experimental.pallas import tpu as pltpu
from jax.experimental.pallas import tpu_sc as plsc

N0 = 50000
N1 = 100000
N2 = 50000
HP = 32
HH = 16
CH = 128
GR = 8
NSUB = 16
EGRAN = NSUB * CH * GR
ACC_ROWS = 102400
GROW = ACC_ROWS - 1
ZB = 256


def _pad_mat(w):
    r, c = w.shape
    return jnp.zeros((HP, HP), jnp.float32).at[:r, :c].set(w)


def _pad_vec_mat(b, w):
    v = b @ w
    return jnp.zeros((HP,), jnp.float32).at[: v.shape[0]].set(v)


def _prep_src(idx):
    e = idx.shape[0]
    ep = -(-e // EGRAN) * EGRAN
    return jnp.concatenate(
        [idx.astype(jnp.int32),
         jnp.zeros((ep - e,), jnp.int32)]).reshape(ep // CH, CH)


def _prep_src8(idx, p0, p1):
    e = idx.shape[0]
    ep = -(-e // EGRAN) * EGRAN
    base = jnp.concatenate(
        [idx.astype(jnp.int32) * 8, jnp.zeros((ep - e,), jnp.int32)])
    return jnp.stack([base + p0, base + p1]).reshape(2, ep // CH, CH)


def _prep_dst(idx):
    e = idx.shape[0]
    ep = -(-e // EGRAN) * EGRAN
    return jnp.concatenate(
        [idx.astype(jnp.int32),
         jnp.full((ep - e,), GROW, jnp.int32)]).reshape(ep // CH, CH)


def _tc_tables_pack(xs, ts, bs):
    n = xs[0].shape[0]
    bn = 400
    nx = len(xs)
    ks = [t.shape[1] for t in ts]
    pad = 128 - sum(ks)

    def body(*refs):
        o = refs[-1]
        parts = []
        for j in range(nx):
            parts.append(jnp.dot(refs[j][...], refs[nx + j][...],
                                 preferred_element_type=jnp.float32)
                         + refs[2 * nx + j][...])
        if pad:
            parts.append(jnp.zeros((bn, pad), jnp.float32))
        o[...] = jnp.concatenate(parts, axis=1)

    return pl.pallas_call(
        body,
        grid=(n // bn,),
        in_specs=([pl.BlockSpec((bn, HP), lambda i: (i, 0))] * nx
                  + [pl.BlockSpec((HP, k), lambda i: (0, 0)) for k in ks]
                  + [pl.BlockSpec((1, k), lambda i: (0, 0)) for k in ks]),
        out_specs=pl.BlockSpec((bn, 128), lambda i: (i, 0)),
        out_shape=jax.ShapeDtypeStruct((n, 128), jnp.float32),
    )(*xs, *ts, *bs)


def _tc_transform2(a, b, u_mat):
    n, k = a.shape[0], u_mat.shape[1]
    bn = 400
    no = k // HH

    def body(a_ref, b_ref, um_ref, *outs):
        h = jax.nn.relu(jnp.concatenate([a_ref[...], b_ref[...]], axis=1))
        y = jnp.dot(h, um_ref[...], preferred_element_type=jnp.float32)
        for j, o in enumerate(outs):
            o[...] = y[:, j * HH:(j + 1) * HH]

    return pl.pallas_call(
        body,
        grid=(n // bn,),
        in_specs=[pl.BlockSpec((bn, HH), lambda i: (i, 0)),
                  pl.BlockSpec((bn, HH), lambda i: (i, 0)),
                  pl.BlockSpec((HP, k), lambda i: (0, 0))],
        out_specs=[pl.BlockSpec((bn, HH), lambda i: (i, 0))] * no,
        out_shape=[jax.ShapeDtypeStruct((n, HH), jnp.float32)] * no,
    )(a, b, u_mat)


def _sc_layer(stages, reduce_out=False):
    ins = []
    meta = []
    for n_rows, jobs in stages:
        jmeta = []
        for job in jobs:
            kind = "shared" if len(job) == 3 else "split"
            rows = job[1].shape[0]
            jmeta.append((kind, len(ins), rows))
            ins.extend(job)
        meta.append((n_rows, jmeta))

    out_type = []
    for n_rows, _ in stages:
        shp = (NSUB, HH) if reduce_out else (n_rows, HH)
        out_type.extend([jax.ShapeDtypeStruct(shp, jnp.float32)] * 2)

    mesh = plsc.VectorSubcoreMesh(core_axis_name="core",
                                  subcore_axis_name="subcore")

    @functools.partial(
        pl.kernel,
        out_type=out_type,
        mesh=mesh,
        compiler_params=pltpu.CompilerParams(use_tc_tiling_on_sc=False),
        scratch_types=[
            pltpu.VMEM_SHARED((ACC_ROWS, HH), jnp.float32),
            pltpu.VMEM((ZB, HH), jnp.float32),
            pltpu.VMEM((2, GR, CH), jnp.int32),
            pltpu.VMEM((2, GR, CH), jnp.int32),
            pltpu.VMEM((GR, CH, HH), jnp.float32),
        ] + [pltpu.SemaphoreType.DMA] * (GR + 3),
    )
    def sc_kernel(*refs):
        n_in = len(ins)
        in_refs = refs[:n_in]
        out_refs = refs[n_in:n_in + 2 * len(stages)]
        scr = refs[n_in + 2 * len(stages):]
        acc, zbuf, srcb, dstb, rowb = scr[:5]
        gsem = scr[5:5 + GR]
        ssem = scr[5 + GR]
        isem = scr[6 + GR:8 + GR]

        cid = lax.axis_index("core")
        sid = lax.axis_index("subcore")

        for si, (n_rows, jmeta) in enumerate(meta):
            per = -(-(n_rows // NSUB) // 8) * 8
            last = n_rows - (NSUB - 1) * per
            rbase = sid * per

            @pl.loop(0, ZB)
            def _(i):
                zbuf[i, :] = jnp.zeros((HH,), jnp.float32)
            def zero_slice(m, rbase=rbase):
                off = 0
                while off < m:
                    step = min(ZB, m - off)
                    pltpu.sync_copy(zbuf.at[pl.ds(0, step)],
                                    acc.at[pl.ds(rbase + off, step)])
                    off += step

            @pl.when(sid < NSUB - 1)
            def _():
                zero_slice(per)

            @pl.when(sid == NSUB - 1)
            def _():
                zero_slice(last)

            plsc.subcore_barrier()

            for (kind, base_i, idx_rows) in jmeta:
                src_ref = in_refs[base_i]
                dst_ref = in_refs[base_i + 1]
                nch = idx_rows // NSUB
                ngr = nch // GR
                crow0 = sid * nch

                def run_job(table_ref, src_slice, ngr=ngr, crow0=crow0,
                            dst_ref=dst_ref):
                    def load_idx(g, b):
                        r = crow0 + g * GR
                        pltpu.async_copy(src_slice(r), srcb.at[b], isem[b])
                        pltpu.async_copy(dst_ref.at[pl.ds(r, GR)],
                                         dstb.at[b], isem[b])

                    def wait_idx(g, b):
                        r = crow0 + g * GR
                        pltpu.make_async_copy(src_slice(r),
                                              srcb.at[b], isem[b]).wait()
                        pltpu.make_async_copy(dst_ref.at[pl.ds(r, GR)],
                                              dstb.at[b], isem[b]).wait()

                    def do_group(g, b):
                        wait_idx(g, b)

                        @pl.when(g + 1 < ngr)
                        def _():
                            load_idx(g + 1, 1 - b)

                        gcps = [pltpu.async_copy(
                                    table_ref.at[srcb.at[b, k]],
                                    rowb.at[k], gsem[k])
                                for k in range(GR)]
                        scps = []
                        for k in range(GR):
                            gcps[k].wait()
                            scps.append(pltpu.async_copy(
                                rowb.at[k], acc.at[dstb.at[b, k]],
                                ssem, add=True))
                        for cp in scps:
                            cp.wait()

                    load_idx(0, 0)
                    if ngr >= 2:
                        @pl.loop(0, ngr // 2)
                        def _(t):
                            do_group(2 * t, 0)
                            do_group(2 * t + 1, 1)
                    if ngr % 2:
                        do_group(ngr - 1, 0)

                if kind == "shared":
                    table = in_refs[base_i + 2]

                    @pl.when(cid == 0)
                    def _():
                        run_job(table,
                                lambda r, s=src_ref: s.at[0, pl.ds(r, GR)])

                    @pl.when(cid == 1)
                    def _():
                        run_job(table,
                                lambda r, s=src_ref: s.at[1, pl.ds(r, GR)])
                else:
                    @pl.when(cid == 0)
                    def _():
                        run_job(in_refs[base_i + 2],
                                lambda r, s=src_ref: s.at[pl.ds(r, GR)])

                    @pl.when(cid == 1)
                    def _():
                        run_job(in_refs[base_i + 3],
                                lambda r, s=src_ref: s.at[pl.ds(r, GR)])

            plsc.subcore_barrier()

            if reduce_out:
                def reduce_slice(m, out_ref, rbase=rbase):
                    s = jnp.zeros((HH,), jnp.float32)
                    off = 0
                    while off < m:
                        step = min(ZB, m - off)
                        pltpu.sync_copy(acc.at[pl.ds(rbase + off, step)],
                                        zbuf.at[pl.ds(0, step)])
                        s = lax.fori_loop(
                            0, step,
                            lambda i, s: s + jnp.maximum(zbuf[i, :], 0.0), s)
                        off += step
                    zbuf[0, :] = s
                    pltpu.sync_copy(zbuf.at[pl.ds(0, 1)],
                                    out_ref.at[pl.ds(sid, 1)])

                def reduce_core(out_ref):
                    @pl.when(sid < NSUB - 1)
                    def _():
                        reduce_slice(per, out_ref)

                    @pl.when(sid == NSUB - 1)
                    def _():
                        reduce_slice(last, out_ref)

                @pl.when(cid == 0)
                def _():
                    reduce_core(out_refs[2 * si])

                @pl.when(cid == 1)
                def _():
                    reduce_core(out_refs[2 * si + 1])
            else:
                def drain(out_ref, rbase=rbase, per=per, last=last):
                    @pl.when(sid < NSUB - 1)
                    def _():
                        pltpu.sync_copy(acc.at[pl.ds(rbase, per)],
                                        out_ref.at[pl.ds(rbase, per)])

                    @pl.when(sid == NSUB - 1)
                    def _():
                        pltpu.sync_copy(acc.at[pl.ds(rbase, last)],
                                        out_ref.at[pl.ds(rbase, last)])

                @pl.when(cid == 0)
                def _():
                    drain(out_refs[2 * si])

                @pl.when(cid == 1)
                def _():
                    drain(out_refs[2 * si + 1])

            plsc.subcore_barrier()

    return sc_kernel(*ins)


def kernel(x_0, x_1, x_2, adjacency_0, adjacency_1, coadjacency_2,
           incidence_1_rows, incidence_1_cols, incidence_2_rows,
           incidence_2_cols,
           W_in0, b_in0, W_in1, b_in1, W_in2, b_in2,
           W00, W10, W01, W11, W12, W22,
           V00, V01, V11, V12, V22,
           Wo0, bo0, Wo1, bo1, Wo2, bo2):
    t0_mat = jnp.concatenate(
        [_pad_mat(W_in0 @ W00), _pad_mat(W_in0 @ W01)], axis=1)
    t0_bias = jnp.concatenate(
        [_pad_vec_mat(b_in0, W00), _pad_vec_mat(b_in0, W01)])[None, :]
    t1_mat = jnp.concatenate(
        [_pad_mat(W_in1 @ W10), _pad_mat(W_in1 @ W11),
         _pad_mat(W_in1 @ W12)], axis=1)
    t1_bias = jnp.concatenate(
        [_pad_vec_mat(b_in1, W10), _pad_vec_mat(b_in1, W11),
         _pad_vec_mat(b_in1, W12)])[None, :]
    t2_mat = _pad_mat(W_in2 @ W22)
    t2_bias = _pad_vec_mat(b_in2, W22)[None, :]

    u0_mat = jnp.concatenate([_pad_mat(V00), _pad_mat(V01)], axis=1)
    u1_mat = jnp.concatenate([_pad_mat(V11), _pad_mat(V12)], axis=1)
    u2_mat = _pad_mat(V22)

    a0_dst = _prep_dst(adjacency_0[0])
    a1_dst = _prep_dst(adjacency_1[0])
    c2_dst = _prep_dst(coadjacency_2[0])
    i1r_src = _prep_src(incidence_1_rows)
    i1r_dst = _prep_dst(incidence_1_rows)
    i1c_dst = _prep_dst(incidence_1_cols)
    i2r_src = _prep_src(incidence_2_rows)
    i2c_dst = _prep_dst(incidence_2_cols)
    a0_src = _prep_src(adjacency_0[1])
    a1_src = _prep_src(adjacency_1[1])
    c2_src = _prep_src(coadjacency_2[1])
    a0_i8 = _prep_src8(adjacency_0[1], 0, 1)
    i1c_i8 = _prep_src8(incidence_1_cols, 0, 1)
    i1r_i8 = _prep_src8(incidence_1_rows, 2, 3)
    a1_i8 = _prep_src8(adjacency_1[1], 2, 3)
    i2r_i8 = _prep_src8(incidence_2_rows, 4, 5)
    c2_i8 = _prep_src8(coadjacency_2[1], 4, 5)

    w02 = _tc_tables_pack([x_0, x_2], [t0_mat, t2_mat], [t0_bias, t2_bias])
    w1 = _tc_tables_pack([x_1], [t1_mat], [t1_bias])
    w02v = w02.reshape(8 * N0, HH)
    w1v = w1.reshape(8 * N1, HH)

    (acc0_0, acc0_1, acc1_0, acc1_1, acc2_0, acc2_1) = _sc_layer([
        (N0, [(a0_i8, a0_dst, w02v),
              (i1c_i8, i1r_dst, w1v)]),
        (N1, [(i1r_i8, i1c_dst, w02v),
              (a1_i8, a1_dst, w1v)]),
        (N2, [(i2r_i8, i2c_dst, w1v),
              (c2_i8, c2_dst, w02v)]),
    ])

    d00_0, d00_1, d01_0, d01_1 = _tc_transform2(acc0_0, acc0_1, u0_mat)
    d11_0, d11_1, d12_0, d12_1 = _tc_transform2(acc1_0, acc1_1, u1_mat)
    d22_0, d22_1 = _tc_transform2(acc2_0, acc2_1, u2_mat)

    (p0_0, p0_1, p1_0, p1_1, p2_0, p2_1) = _sc_layer([
        (N0, [(a0_src, a0_dst, d00_0, d00_1)]),
        (N1, [(i1r_src, i1c_dst, d01_0, d01_1),
              (a1_src, a1_dst, d11_0, d11_1)]),
        (N2, [(i2r_src, i2c_dst, d12_0, d12_1),
              (c2_src, c2_dst, d22_0, d22_1)]),
    ], reduce_out=True)

    def head(p_a, p_b, n, wo, bo):
        s = jnp.concatenate([p_a.sum(0), p_b.sum(0)])[None, :]
        wo_p = jnp.zeros((HP, 1), jnp.float32).at[:wo.shape[0]].set(wo)
        return (s @ wo_p)[0] / n + bo

    return (head(p0_0, p0_1, N0, Wo0, bo0) + head(p1_0, p1_1, N1, Wo1, bo1)
            + head(p2_0, p2_1, N2, Wo2, bo2))


# speedup vs baseline: 12.5860x; 1.5862x over previous
import functools

import jax
import jax.numpy as jnp
from jax import lax
from jax.experimental import pallas as pl
from jax.experimental.pallas import tpu as pltpu
from jax.experimental.pallas import tpu_sc as plsc

N0 = 50000
N1 = 100000
N2 = 50000
HP = 32
HH = 16
CH = 128
GR = 8
NSUB = 16
EGRAN = NSUB * CH * GR
ACC_ROWS = 102400
GROW = ACC_ROWS - 1
ZB = 256


def _pad_mat(w):
    r, c = w.shape
    return jnp.zeros((HP, HP), jnp.float32).at[:r, :c].set(w)


def _pad_vec_mat(b, w):
    v = b @ w
    return jnp.zeros((HP,), jnp.float32).at[: v.shape[0]].set(v)


def _kron8(block):
    return jnp.kron(jnp.eye(8, dtype=jnp.float32), block)


def _halves(mat, bias=None):
    k = mat.shape[1]
    mats = [_kron8(mat[:, j * HH:(j + 1) * HH]) for j in range(k // HH)]
    if bias is None:
        return mats
    bs = [jnp.tile(bias[j * HH:(j + 1) * HH], 8)[None, :]
          for j in range(k // HH)]
    return mats, bs


def _prep_src(idx):
    e = idx.shape[0]
    ep = -(-e // EGRAN) * EGRAN
    return jnp.concatenate(
        [idx.astype(jnp.int32),
         jnp.zeros((ep - e,), jnp.int32)]).reshape(ep // CH, CH)


def _prep_dst(idx):
    e = idx.shape[0]
    ep = -(-e // EGRAN) * EGRAN
    return jnp.concatenate(
        [idx.astype(jnp.int32),
         jnp.full((ep - e,), GROW, jnp.int32)]).reshape(ep // CH, CH)


def _tc_kron_tables(xp, mats, biases):
    m, kin = xp.shape
    bn = m // 8
    no = len(mats)

    def body(*refs):
        x = refs[0][...]
        outs = refs[1 + 2 * no:]
        for i, o in enumerate(outs):
            o[...] = jnp.dot(x, refs[1 + i][...],
                             preferred_element_type=jnp.float32) \
                + refs[1 + no + i][...]

    return pl.pallas_call(
        body,
        grid=(m // bn,),
        in_specs=([pl.BlockSpec((bn, kin), lambda i: (i, 0))]
                  + [pl.BlockSpec((kin, 128), lambda i: (0, 0))] * no
                  + [pl.BlockSpec((1, 128), lambda i: (0, 0))] * no),
        out_specs=[pl.BlockSpec((bn, 128), lambda i: (i, 0))] * no,
        out_shape=[jax.ShapeDtypeStruct((m, 128), jnp.float32)] * no,
    )(xp, *mats, *biases)


def _tc_kron_mid(a0p, a1p, mats0, mats1):
    m = a0p.shape[0]
    bn = m // 8
    no = len(mats0)

    def body(*refs):
        h0 = jax.nn.relu(refs[0][...])
        h1 = jax.nn.relu(refs[1][...])
        outs = refs[2 + 2 * no:]
        for i, o in enumerate(outs):
            o[...] = jnp.dot(h0, refs[2 + i][...],
                             preferred_element_type=jnp.float32) \
                + jnp.dot(h1, refs[2 + no + i][...],
                          preferred_element_type=jnp.float32)

    return pl.pallas_call(
        body,
        grid=(m // bn,),
        in_specs=([pl.BlockSpec((bn, 128), lambda i: (i, 0))] * 2
                  + [pl.BlockSpec((128, 128), lambda i: (0, 0))] * 2 * no),
        out_specs=[pl.BlockSpec((bn, 128), lambda i: (i, 0))] * no,
        out_shape=[jax.ShapeDtypeStruct((m, 128), jnp.float32)] * no,
    )(a0p, a1p, *mats0, *mats1)


def _sc_layer(stages, reduce_out=False):
    ins = []
    meta = []
    for n_rows, n_pad, jobs in stages:
        jmeta = []
        for job in jobs:
            jmeta.append((len(ins), job[1].shape[0]))
            ins.extend(job)
        meta.append((n_rows, jmeta))

    out_type = []
    for n_rows, n_pad, _ in stages:
        shp = (NSUB, HH) if reduce_out else (n_pad, HH)
        out_type.extend([jax.ShapeDtypeStruct(shp, jnp.float32)] * 2)

    mesh = plsc.VectorSubcoreMesh(core_axis_name="core",
                                  subcore_axis_name="subcore")

    @functools.partial(
        pl.kernel,
        out_type=out_type,
        mesh=mesh,
        compiler_params=pltpu.CompilerParams(use_tc_tiling_on_sc=False),
        scratch_types=[
            pltpu.VMEM_SHARED((ACC_ROWS, HH), jnp.float32),
            pltpu.VMEM((ZB, HH), jnp.float32),
            pltpu.VMEM((2, GR, CH), jnp.int32),
            pltpu.VMEM((2, GR, CH), jnp.int32),
            pltpu.VMEM((GR, CH, HH), jnp.float32),
        ] + [pltpu.SemaphoreType.DMA] * (GR + 3),
    )
    def sc_kernel(*refs):
        n_in = len(ins)
        in_refs = refs[:n_in]
        out_refs = refs[n_in:n_in + 2 * len(stages)]
        scr = refs[n_in + 2 * len(stages):]
        acc, zbuf, srcb, dstb, rowb = scr[:5]
        gsem = scr[5:5 + GR]
        ssem = scr[5 + GR]
        isem = scr[6 + GR:8 + GR]

        cid = lax.axis_index("core")
        sid = lax.axis_index("subcore")

        for si, (n_rows, jmeta) in enumerate(meta):
            per = -(-(n_rows // NSUB) // 8) * 8
            last = n_rows - (NSUB - 1) * per
            rbase = sid * per

            @pl.loop(0, ZB)
            def _(i):
                zbuf[i, :] = jnp.zeros((HH,), jnp.float32)

            def zero_slice(m, rbase=rbase):
                off = 0
                while off < m:
                    step = min(ZB, m - off)
                    pltpu.sync_copy(zbuf.at[pl.ds(0, step)],
                                    acc.at[pl.ds(rbase + off, step)])
                    off += step

            @pl.when(sid < NSUB - 1)
            def _():
                zero_slice(per)

            @pl.when(sid == NSUB - 1)
            def _():
                zero_slice(last)

            plsc.subcore_barrier()

            for (base_i, idx_rows) in jmeta:
                src_ref = in_refs[base_i]
                dst_ref = in_refs[base_i + 1]
                nch = idx_rows // NSUB
                ngr = nch // GR
                crow0 = sid * nch

                def run_job(table_ref, ngr=ngr, crow0=crow0,
                            src_ref=src_ref, dst_ref=dst_ref):
                    def load_idx(g, b):
                        r = crow0 + g * GR
                        pltpu.async_copy(src_ref.at[pl.ds(r, GR)],
                                         srcb.at[b], isem[b])
                        pltpu.async_copy(dst_ref.at[pl.ds(r, GR)],
                                         dstb.at[b], isem[b])

                    def wait_idx(g, b):
                        r = crow0 + g * GR
                        pltpu.make_async_copy(src_ref.at[pl.ds(r, GR)],
                                              srcb.at[b], isem[b]).wait()
                        pltpu.make_async_copy(dst_ref.at[pl.ds(r, GR)],
                                              dstb.at[b], isem[b]).wait()

                    def do_group(g, b):
                        wait_idx(g, b)

                        @pl.when(g + 1 < ngr)
                        def _():
                            load_idx(g + 1, 1 - b)

                        gcps = [pltpu.async_copy(
                                    table_ref.at[srcb.at[b, k]],
                                    rowb.at[k], gsem[k])
                                for k in range(GR)]
                        scps = []
                        for k in range(GR):
                            gcps[k].wait()
                            scps.append(pltpu.async_copy(
                                rowb.at[k], acc.at[dstb.at[b, k]],
                                ssem, add=True))
                        for cp in scps:
                            cp.wait()

                    load_idx(0, 0)
                    if ngr >= 2:
                        @pl.loop(0, ngr // 2)
                        def _(t):
                            do_group(2 * t, 0)
                            do_group(2 * t + 1, 1)
                    if ngr % 2:
                        do_group(ngr - 1, 0)

                @pl.when(cid == 0)
                def _():
                    run_job(in_refs[base_i + 2])

                @pl.when(cid == 1)
                def _():
                    run_job(in_refs[base_i + 3])

            plsc.subcore_barrier()

            if reduce_out:
                def reduce_slice(m, out_ref, rbase=rbase):
                    s = jnp.zeros((HH,), jnp.float32)
                    off = 0
                    while off < m:
                        step = min(ZB, m - off)
                        pltpu.sync_copy(acc.at[pl.ds(rbase + off, step)],
                                        zbuf.at[pl.ds(0, step)])
                        s = lax.fori_loop(
                            0, step,
                            lambda i, s: s + jnp.maximum(zbuf[i, :], 0.0), s)
                        off += step
                    zbuf[0, :] = s
                    pltpu.sync_copy(zbuf.at[pl.ds(0, 1)],
                                    out_ref.at[pl.ds(sid, 1)])

                def reduce_core(out_ref):
                    @pl.when(sid < NSUB - 1)
                    def _():
                        reduce_slice(per, out_ref)

                    @pl.when(sid == NSUB - 1)
                    def _():
                        reduce_slice(last, out_ref)

                @pl.when(cid == 0)
                def _():
                    reduce_core(out_refs[2 * si])

                @pl.when(cid == 1)
                def _():
                    reduce_core(out_refs[2 * si + 1])
            else:
                def drain(out_ref, rbase=rbase, per=per, last=last):
                    @pl.when(sid < NSUB - 1)
                    def _():
                        pltpu.sync_copy(acc.at[pl.ds(rbase, per)],
                                        out_ref.at[pl.ds(rbase, per)])

                    @pl.when(sid == NSUB - 1)
                    def _():
                        pltpu.sync_copy(acc.at[pl.ds(rbase, last)],
                                        out_ref.at[pl.ds(rbase, last)])

                @pl.when(cid == 0)
                def _():
                    drain(out_refs[2 * si])

                @pl.when(cid == 1)
                def _():
                    drain(out_refs[2 * si + 1])

            plsc.subcore_barrier()

    return sc_kernel(*ins)


def kernel(x_0, x_1, x_2, adjacency_0, adjacency_1, coadjacency_2,
           incidence_1_rows, incidence_1_cols, incidence_2_rows,
           incidence_2_cols,
           W_in0, b_in0, W_in1, b_in1, W_in2, b_in2,
           W00, W10, W01, W11, W12, W22,
           V00, V01, V11, V12, V22,
           Wo0, bo0, Wo1, bo1, Wo2, bo2):
    t0_mats, t0_bs = _halves(
        jnp.concatenate([_pad_mat(W_in0 @ W00), _pad_mat(W_in0 @ W01)], 1),
        jnp.concatenate([_pad_vec_mat(b_in0, W00), _pad_vec_mat(b_in0, W01)]))
    t1_mats, t1_bs = _halves(
        jnp.concatenate([_pad_mat(W_in1 @ W10), _pad_mat(W_in1 @ W11),
                         _pad_mat(W_in1 @ W12)], 1),
        jnp.concatenate([_pad_vec_mat(b_in1, W10), _pad_vec_mat(b_in1, W11),
                         _pad_vec_mat(b_in1, W12)]))
    t2_mats, t2_bs = _halves(_pad_mat(W_in2 @ W22),
                             _pad_vec_mat(b_in2, W22))

    def mid_mats(u):
        k = u.shape[1]
        m0 = [_kron8(u[:HH, j * HH:(j + 1) * HH]) for j in range(k // HH)]
        m1 = [_kron8(u[HH:, j * HH:(j + 1) * HH]) for j in range(k // HH)]
        return m0, m1

    u0_m0, u0_m1 = mid_mats(
        jnp.concatenate([_pad_mat(V00), _pad_mat(V01)], 1))
    u1_m0, u1_m1 = mid_mats(
        jnp.concatenate([_pad_mat(V11), _pad_mat(V12)], 1))
    u2_m0, u2_m1 = mid_mats(_pad_mat(V22))

    a0_dst, a0_src = _prep_dst(adjacency_0[0]), _prep_src(adjacency_0[1])
    a1_dst, a1_src = _prep_dst(adjacency_1[0]), _prep_src(adjacency_1[1])
    c2_dst, c2_src = _prep_dst(coadjacency_2[0]), _prep_src(coadjacency_2[1])
    i1r_src = _prep_src(incidence_1_rows)
    i1r_dst = _prep_dst(incidence_1_rows)
    i1c_src = _prep_src(incidence_1_cols)
    i1c_dst = _prep_dst(incidence_1_cols)
    i2r_src = _prep_src(incidence_2_rows)
    i2c_dst = _prep_dst(incidence_2_cols)

    n0p, n1p, n2p = 51200, 102400, 51200
    v = lambda a: a.reshape(a.shape[0] * 8, HH)

    def xpack(x, n, np_):
        xp = x.reshape(n // 8, 8 * HP)
        return jnp.concatenate(
            [xp, jnp.zeros((np_ // 8 - n // 8, 8 * HP), jnp.float32)])

    g00_0, g00_1, g01_0, g01_1 = map(v, _tc_kron_tables(
        xpack(x_0, N0, n0p), t0_mats, t0_bs))
    g10_0, g10_1, g11_0, g11_1, g12_0, g12_1 = map(v, _tc_kron_tables(
        xpack(x_1, N1, n1p), t1_mats, t1_bs))
    g22_0, g22_1 = map(v, _tc_kron_tables(
        xpack(x_2, N2, n2p), t2_mats, t2_bs))

    (acc0_0, acc0_1, acc1_0, acc1_1, acc2_0, acc2_1) = _sc_layer([
        (N0, n0p, [(a0_src, a0_dst, g00_0, g00_1),
                   (i1c_src, i1r_dst, g10_0, g10_1)]),
        (N1, n1p, [(i1r_src, i1c_dst, g01_0, g01_1),
                   (a1_src, a1_dst, g11_0, g11_1)]),
        (N2, n2p, [(i2r_src, i2c_dst, g12_0, g12_1),
                   (c2_src, c2_dst, g22_0, g22_1)]),
    ])

    p = lambda a: a.reshape(a.shape[0] // 8, 128)
    d00_0, d00_1, d01_0, d01_1 = map(v, _tc_kron_mid(
        p(acc0_0), p(acc0_1), u0_m0, u0_m1))
    d11_0, d11_1, d12_0, d12_1 = map(v, _tc_kron_mid(
        p(acc1_0), p(acc1_1), u1_m0, u1_m1))
    d22_0, d22_1 = map(v, _tc_kron_mid(
        p(acc2_0), p(acc2_1), u2_m0, u2_m1))

    (p0_0, p0_1, p1_0, p1_1, p2_0, p2_1) = _sc_layer([
        (N0, n0p, [(a0_src, a0_dst, d00_0, d00_1)]),
        (N1, n1p, [(i1r_src, i1c_dst, d01_0, d01_1),
                   (a1_src, a1_dst, d11_0, d11_1)]),
        (N2, n2p, [(i2r_src, i2c_dst, d12_0, d12_1),
                   (c2_src, c2_dst, d22_0, d22_1)]),
    ], reduce_out=True)

    def head(p_a, p_b, n, wo, bo):
        s = jnp.concatenate([p_a.sum(0), p_b.sum(0)])[None, :]
        wo_p = jnp.zeros((HP, 1), jnp.float32).at[:wo.shape[0]].set(wo)
        return (s @ wo_p)[0] / n + bo

    return (head(p0_0, p0_1, N0, Wo0, bo0) + head(p1_0, p1_1, N1, Wo1, bo1)
            + head(p2_0, p2_1, N2, Wo2, bo2))


# speedup vs baseline: 13.7361x; 1.0914x over previous
import functools

import jax
import jax.numpy as jnp
from jax import lax
from jax.experimental import pallas as pl
from jax.experimental.pallas import tpu as pltpu
from jax.experimental.pallas import tpu_sc as plsc

N0 = 50000
N1 = 100000
N2 = 50000
HP = 32
HH = 16
CH = 128
GR = 8
NSUB = 16
EGRAN = NSUB * CH * GR
ACC_ROWS = 102400
GROW = ACC_ROWS - 1
TOFF = 51200
ZB = 448


def _pad_mat(w):
    r, c = w.shape
    return jnp.zeros((HP, HP), jnp.float32).at[:r, :c].set(w)


def _pad_vec_mat(b, w):
    v = b @ w
    return jnp.zeros((HP,), jnp.float32).at[: v.shape[0]].set(v)


def _kron8(block):
    return jnp.kron(jnp.eye(8, dtype=jnp.float32), block)


def _halves(mat, bias=None):
    k = mat.shape[1]
    mats = [_kron8(mat[:, j * HH:(j + 1) * HH]) for j in range(k // HH)]
    if bias is None:
        return mats
    bs = [jnp.tile(bias[j * HH:(j + 1) * HH], 8)[None, :]
          for j in range(k // HH)]
    return mats, bs


def _prep_src(idx, offset=0):
    e = idx.shape[0]
    ep = -(-e // EGRAN) * EGRAN
    return jnp.concatenate(
        [idx.astype(jnp.int32) + offset,
         jnp.full((ep - e,), offset, jnp.int32)]).reshape(ep // CH, CH)


def _prep_dst(idx):
    e = idx.shape[0]
    ep = -(-e // EGRAN) * EGRAN
    return jnp.concatenate(
        [idx.astype(jnp.int32),
         jnp.full((ep - e,), GROW, jnp.int32)]).reshape(ep // CH, CH)


def _tc_kron_tables(xp, mats, biases):
    m, kin = xp.shape
    bn = m // 8
    no = len(mats)

    def body(*refs):
        x = refs[0][...]
        outs = refs[1 + 2 * no:]
        for i, o in enumerate(outs):
            o[...] = jnp.dot(x, refs[1 + i][...],
                             preferred_element_type=jnp.float32) \
                + refs[1 + no + i][...]

    return pl.pallas_call(
        body,
        grid=(m // bn,),
        in_specs=([pl.BlockSpec((bn, kin), lambda i: (i, 0))]
                  + [pl.BlockSpec((kin, 128), lambda i: (0, 0))] * no
                  + [pl.BlockSpec((1, 128), lambda i: (0, 0))] * no),
        out_specs=[pl.BlockSpec((bn, 128), lambda i: (i, 0))] * no,
        out_shape=[jax.ShapeDtypeStruct((m, 128), jnp.float32)] * no,
    )(xp, *mats, *biases)


def _tc_kron_mid(a0p, a1p, mats0, mats1):
    m = a0p.shape[0]
    bn = m // 8
    no = len(mats0)

    def body(*refs):
        h0 = jax.nn.relu(refs[0][...])
        h1 = jax.nn.relu(refs[1][...])
        outs = refs[2 + 2 * no:]
        for i, o in enumerate(outs):
            o[...] = jnp.dot(h0, refs[2 + i][...],
                             preferred_element_type=jnp.float32) \
                + jnp.dot(h1, refs[2 + no + i][...],
                          preferred_element_type=jnp.float32)

    return pl.pallas_call(
        body,
        grid=(m // bn,),
        in_specs=([pl.BlockSpec((bn, 128), lambda i: (i, 0))] * 2
                  + [pl.BlockSpec((128, 128), lambda i: (0, 0))] * 2 * no),
        out_specs=[pl.BlockSpec((bn, 128), lambda i: (i, 0))] * no,
        out_shape=[jax.ShapeDtypeStruct((m, 128), jnp.float32)] * no,
    )(a0p, a1p, *mats0, *mats1)


def _sc_layer(stages, reduce_out=False):
    ins = []
    meta = []
    for n_rows, n_pad, jobs in stages:
        jmeta = []
        for job in jobs:
            jmeta.append((len(ins), job[1].shape[0], job[4]))
            ins.extend(job[:4])
        meta.append((n_rows, jmeta))

    out_type = []
    for n_rows, n_pad, _ in stages:
        shp = (NSUB, HH) if reduce_out else (n_pad, HH)
        out_type.extend([jax.ShapeDtypeStruct(shp, jnp.float32)] * 2)

    mesh = plsc.VectorSubcoreMesh(core_axis_name="core",
                                  subcore_axis_name="subcore")

    @functools.partial(
        pl.kernel,
        out_type=out_type,
        mesh=mesh,
        compiler_params=pltpu.CompilerParams(use_tc_tiling_on_sc=False),
        scratch_types=[
            pltpu.VMEM_SHARED((ACC_ROWS, HH), jnp.float32),
            pltpu.VMEM((ZB, HH), jnp.float32),
            pltpu.VMEM((2, GR, CH), jnp.int32),
            pltpu.VMEM((2, GR, CH), jnp.int32),
            pltpu.VMEM((GR, CH, HH), jnp.float32),
        ] + [pltpu.SemaphoreType.DMA] * (GR + 3),
    )
    def sc_kernel(*refs):
        n_in = len(ins)
        in_refs = refs[:n_in]
        out_refs = refs[n_in:n_in + 2 * len(stages)]
        scr = refs[n_in + 2 * len(stages):]
        acc, zbuf, srcb, dstb, rowb = scr[:5]
        gsem = scr[5:5 + GR]
        ssem = scr[5 + GR]
        isem = scr[6 + GR:8 + GR]

        cid = lax.axis_index("core")
        sid = lax.axis_index("subcore")

        for si, (n_rows, jmeta) in enumerate(meta):
            per = -(-(n_rows // NSUB) // 8) * 8
            last = n_rows - (NSUB - 1) * per
            rbase = sid * per

            @pl.loop(0, ZB)
            def _(i):
                zbuf[i, :] = jnp.zeros((HH,), jnp.float32)

            def zero_slice(m, rbase=rbase):
                nfull, rem = m // ZB, m % ZB
                if nfull:
                    @pl.loop(0, nfull)
                    def _(i):
                        pltpu.sync_copy(
                            zbuf, acc.at[pl.ds(rbase + i * ZB, ZB)])
                if rem:
                    pltpu.sync_copy(
                        zbuf.at[pl.ds(0, rem)],
                        acc.at[pl.ds(rbase + nfull * ZB, rem)])

            @pl.when(sid < NSUB - 1)
            def _():
                zero_slice(per)

            @pl.when(sid == NSUB - 1)
            def _():
                zero_slice(last)

            for (base_i, idx_rows, cached) in jmeta:
                if not cached:
                    continue

                def tload(m, t_ref, rbase=rbase):
                    pltpu.sync_copy(t_ref.at[pl.ds(rbase, m)],
                                    acc.at[pl.ds(TOFF + rbase, m)])

                def tload_core(t_ref):
                    @pl.when(sid < NSUB - 1)
                    def _():
                        tload(per, t_ref)

                    @pl.when(sid == NSUB - 1)
                    def _():
                        tload(last, t_ref)

                @pl.when(cid == 0)
                def _():
                    tload_core(in_refs[base_i + 2])

                @pl.when(cid == 1)
                def _():
                    tload_core(in_refs[base_i + 3])

            plsc.subcore_barrier()

            for (base_i, idx_rows, cached) in jmeta:
                src_ref = in_refs[base_i]
                dst_ref = in_refs[base_i + 1]
                nch = idx_rows // NSUB
                ngr = nch // GR
                crow0 = sid * nch

                def run_job(table_ref, ngr=ngr, crow0=crow0,
                            src_ref=src_ref, dst_ref=dst_ref):
                    def load_idx(g, b):
                        r = crow0 + g * GR
                        pltpu.async_copy(src_ref.at[pl.ds(r, GR)],
                                         srcb.at[b], isem[b])
                        pltpu.async_copy(dst_ref.at[pl.ds(r, GR)],
                                         dstb.at[b], isem[b])

                    def wait_idx(g, b):
                        r = crow0 + g * GR
                        pltpu.make_async_copy(src_ref.at[pl.ds(r, GR)],
                                              srcb.at[b], isem[b]).wait()
                        pltpu.make_async_copy(dst_ref.at[pl.ds(r, GR)],
                                              dstb.at[b], isem[b]).wait()

                    def do_group(g, b):
                        wait_idx(g, b)

                        @pl.when(g + 1 < ngr)
                        def _():
                            load_idx(g + 1, 1 - b)

                        gcps = [pltpu.async_copy(
                                    table_ref.at[srcb.at[b, k]],
                                    rowb.at[k], gsem[k])
                                for k in range(GR)]
                        scps = []
                        for k in range(GR):
                            gcps[k].wait()
                            scps.append(pltpu.async_copy(
                                rowb.at[k], acc.at[dstb.at[b, k]],
                                ssem, add=True))
                        for cp in scps:
                            cp.wait()

                    load_idx(0, 0)
                    if ngr >= 2:
                        @pl.loop(0, ngr // 2)
                        def _(t):
                            do_group(2 * t, 0)
                            do_group(2 * t + 1, 1)
                    if ngr % 2:
                        do_group(ngr - 1, 0)

                if cached:
                    run_job(acc)
                else:
                    @pl.when(cid == 0)
                    def _():
                        run_job(in_refs[base_i + 2])

                    @pl.when(cid == 1)
                    def _():
                        run_job(in_refs[base_i + 3])

            plsc.subcore_barrier()

            if reduce_out:
                def reduce_slice(m, out_ref, rbase=rbase):
                    nfull, rem = m // ZB, m % ZB
                    rowb[0, 0, :] = jnp.zeros((HH,), jnp.float32)
                    if nfull:
                        @pl.loop(0, nfull)
                        def _(i):
                            pltpu.sync_copy(
                                acc.at[pl.ds(rbase + i * ZB, ZB)], zbuf)
                            rowb[0, 0, :] += lax.fori_loop(
                                0, ZB,
                                lambda j, s: s + jnp.maximum(zbuf[j, :], 0.0),
                                jnp.zeros((HH,), jnp.float32))
                    if rem:
                        pltpu.sync_copy(
                            acc.at[pl.ds(rbase + nfull * ZB, rem)],
                            zbuf.at[pl.ds(0, rem)])
                        rowb[0, 0, :] += lax.fori_loop(
                            0, rem,
                            lambda j, s: s + jnp.maximum(zbuf[j, :], 0.0),
                            jnp.zeros((HH,), jnp.float32))
                    pltpu.sync_copy(rowb.at[0, pl.ds(0, 1)],
                                    out_ref.at[pl.ds(sid, 1)])

                def reduce_core(out_ref):
                    @pl.when(sid < NSUB - 1)
                    def _():
                        reduce_slice(per, out_ref)

                    @pl.when(sid == NSUB - 1)
                    def _():
                        reduce_slice(last, out_ref)

                @pl.when(cid == 0)
                def _():
                    reduce_core(out_refs[2 * si])

                @pl.when(cid == 1)
                def _():
                    reduce_core(out_refs[2 * si + 1])
            else:
                def drain(out_ref, rbase=rbase, per=per, last=last):
                    @pl.when(sid < NSUB - 1)
                    def _():
                        pltpu.sync_copy(acc.at[pl.ds(rbase, per)],
                                        out_ref.at[pl.ds(rbase, per)])

                    @pl.when(sid == NSUB - 1)
                    def _():
                        pltpu.sync_copy(acc.at[pl.ds(rbase, last)],
                                        out_ref.at[pl.ds(rbase, last)])

                @pl.when(cid == 0)
                def _():
                    drain(out_refs[2 * si])

                @pl.when(cid == 1)
                def _():
                    drain(out_refs[2 * si + 1])

            plsc.subcore_barrier()

    return sc_kernel(*ins)


def kernel(x_0, x_1, x_2, adjacency_0, adjacency_1, coadjacency_2,
           incidence_1_rows, incidence_1_cols, incidence_2_rows,
           incidence_2_cols,
           W_in0, b_in0, W_in1, b_in1, W_in2, b_in2,
           W00, W10, W01, W11, W12, W22,
           V00, V01, V11, V12, V22,
           Wo0, bo0, Wo1, bo1, Wo2, bo2):
    t0_mats, t0_bs = _halves(
        jnp.concatenate([_pad_mat(W_in0 @ W00), _pad_mat(W_in0 @ W01)], 1),
        jnp.concatenate([_pad_vec_mat(b_in0, W00), _pad_vec_mat(b_in0, W01)]))
    t1_mats, t1_bs = _halves(
        jnp.concatenate([_pad_mat(W_in1 @ W10), _pad_mat(W_in1 @ W11),
                         _pad_mat(W_in1 @ W12)], 1),
        jnp.concatenate([_pad_vec_mat(b_in1, W10), _pad_vec_mat(b_in1, W11),
                         _pad_vec_mat(b_in1, W12)]))
    t2_mats, t2_bs = _halves(_pad_mat(W_in2 @ W22),
                             _pad_vec_mat(b_in2, W22))

    def mid_mats(u):
        k = u.shape[1]
        m0 = [_kron8(u[:HH, j * HH:(j + 1) * HH]) for j in range(k // HH)]
        m1 = [_kron8(u[HH:, j * HH:(j + 1) * HH]) for j in range(k // HH)]
        return m0, m1

    u0_m0, u0_m1 = mid_mats(
        jnp.concatenate([_pad_mat(V00), _pad_mat(V01)], 1))
    u1_m0, u1_m1 = mid_mats(
        jnp.concatenate([_pad_mat(V11), _pad_mat(V12)], 1))
    u2_m0, u2_m1 = mid_mats(_pad_mat(V22))

    a0_dst = _prep_dst(adjacency_0[0])
    a0_srcT = _prep_src(adjacency_0[1], TOFF)
    a1_dst, a1_src = _prep_dst(adjacency_1[0]), _prep_src(adjacency_1[1])
    c2_dst = _prep_dst(coadjacency_2[0])
    c2_srcT = _prep_src(coadjacency_2[1], TOFF)
    i1r_src = _prep_src(incidence_1_rows)
    i1r_dst = _prep_dst(incidence_1_rows)
    i1c_src = _prep_src(incidence_1_cols)
    i1c_dst = _prep_dst(incidence_1_cols)
    i2r_src = _prep_src(incidence_2_rows)
    i2c_dst = _prep_dst(incidence_2_cols)

    n0p, n1p, n2p = 51200, 102400, 51200
    v = lambda a: a.reshape(a.shape[0] * 8, HH)

    def xpack(x, n, np_):
        xp = x.reshape(n // 8, 8 * HP)
        return jnp.concatenate(
            [xp, jnp.zeros((np_ // 8 - n // 8, 8 * HP), jnp.float32)])

    g00_0, g00_1, g01_0, g01_1 = map(v, _tc_kron_tables(
        xpack(x_0, N0, n0p), t0_mats, t0_bs))
    g10_0, g10_1, g11_0, g11_1, g12_0, g12_1 = map(v, _tc_kron_tables(
        xpack(x_1, N1, n1p), t1_mats, t1_bs))
    g22_0, g22_1 = map(v, _tc_kron_tables(
        xpack(x_2, N2, n2p), t2_mats, t2_bs))

    (acc0_0, acc0_1, acc1_0, acc1_1, acc2_0, acc2_1) = _sc_layer([
        (N0, n0p, [(a0_srcT, a0_dst, g00_0, g00_1, True),
                   (i1c_src, i1r_dst, g10_0, g10_1, False)]),
        (N1, n1p, [(i1r_src, i1c_dst, g01_0, g01_1, False),
                   (a1_src, a1_dst, g11_0, g11_1, False)]),
        (N2, n2p, [(i2r_src, i2c_dst, g12_0, g12_1, False),
                   (c2_srcT, c2_dst, g22_0, g22_1, True)]),
    ])

    p = lambda a: a.reshape(a.shape[0] // 8, 128)
    d00_0, d00_1, d01_0, d01_1 = map(v, _tc_kron_mid(
        p(acc0_0), p(acc0_1), u0_m0, u0_m1))
    d11_0, d11_1, d12_0, d12_1 = map(v, _tc_kron_mid(
        p(acc1_0), p(acc1_1), u1_m0, u1_m1))
    d22_0, d22_1 = map(v, _tc_kron_mid(
        p(acc2_0), p(acc2_1), u2_m0, u2_m1))

    (p0_0, p0_1, p1_0, p1_1, p2_0, p2_1) = _sc_layer([
        (N0, n0p, [(a0_srcT, a0_dst, d00_0, d00_1, True)]),
        (N1, n1p, [(i1r_src, i1c_dst, d01_0, d01_1, False),
                   (a1_src, a1_dst, d11_0, d11_1, False)]),
        (N2, n2p, [(i2r_src, i2c_dst, d12_0, d12_1, False),
                   (c2_srcT, c2_dst, d22_0, d22_1, True)]),
    ], reduce_out=True)

    def head(p_a, p_b, n, wo, bo):
        s = jnp.concatenate([p_a.sum(0), p_b.sum(0)])[None, :]
        wo_p = jnp.zeros((HP, 1), jnp.float32).at[:wo.shape[0]].set(wo)
        return (s @ wo_p)[0] / n + bo

    return (head(p0_0, p0_1, N0, Wo0, bo0) + head(p1_0, p1_1, N1, Wo1, bo1)
            + head(p2_0, p2_1, N2, Wo2, bo2))


# speedup vs baseline: 13.7402x; 1.0003x over previous
import functools

import jax
import jax.numpy as jnp
from jax import lax
from jax.experimental import pallas as pl
from jax.experimental.pallas import tpu as pltpu
from jax.experimental.pallas import tpu_sc as plsc

N0 = 50000
N1 = 100000
N2 = 50000
HP = 32
HH = 16
CH = 128
CR = 1
GR = 8
NSUB = 16
EGRAN = NSUB * CH * CR * GR
ACC_ROWS = 102400
GROW = ACC_ROWS - 1
TOFF = 51200
ZB = 448


def _pad_mat(w):
    r, c = w.shape
    return jnp.zeros((HP, HP), jnp.float32).at[:r, :c].set(w)


def _pad_vec_mat(b, w):
    v = b @ w
    return jnp.zeros((HP,), jnp.float32).at[: v.shape[0]].set(v)


def _kron8(block):
    return jnp.kron(jnp.eye(8, dtype=jnp.float32), block)


def _halves(mat, bias=None):
    k = mat.shape[1]
    mats = [_kron8(mat[:, j * HH:(j + 1) * HH]) for j in range(k // HH)]
    if bias is None:
        return mats
    bs = [jnp.tile(bias[j * HH:(j + 1) * HH], 8)[None, :]
          for j in range(k // HH)]
    return mats, bs


def _prep_src(idx, offset=0):
    e = idx.shape[0]
    ep = -(-e // EGRAN) * EGRAN
    return jnp.concatenate(
        [idx.astype(jnp.int32) + offset,
         jnp.full((ep - e,), offset, jnp.int32)]).reshape(ep // CH, CH)


def _prep_dst(idx):
    e = idx.shape[0]
    ep = -(-e // EGRAN) * EGRAN
    return jnp.concatenate(
        [idx.astype(jnp.int32),
         jnp.full((ep - e,), GROW, jnp.int32)]).reshape(ep // CH, CH)


def _tc_kron_tables(xp, mats, biases):
    m, kin = xp.shape
    bn = m // 8
    no = len(mats)

    def body(*refs):
        x = refs[0][...]
        outs = refs[1 + 2 * no:]
        for i, o in enumerate(outs):
            o[...] = jnp.dot(x, refs[1 + i][...],
                             preferred_element_type=jnp.float32) \
                + refs[1 + no + i][...]

    return pl.pallas_call(
        body,
        grid=(m // bn,),
        in_specs=([pl.BlockSpec((bn, kin), lambda i: (i, 0))]
                  + [pl.BlockSpec((kin, 128), lambda i: (0, 0))] * no
                  + [pl.BlockSpec((1, 128), lambda i: (0, 0))] * no),
        out_specs=[pl.BlockSpec((bn, 128), lambda i: (i, 0))] * no,
        out_shape=[jax.ShapeDtypeStruct((m, 128), jnp.float32)] * no,
    )(xp, *mats, *biases)


def _tc_kron_mid(a0p, a1p, mats0, mats1):
    m = a0p.shape[0]
    bn = m // 8
    no = len(mats0)

    def body(*refs):
        h0 = jax.nn.relu(refs[0][...])
        h1 = jax.nn.relu(refs[1][...])
        outs = refs[2 + 2 * no:]
        for i, o in enumerate(outs):
            o[...] = jnp.dot(h0, refs[2 + i][...],
                             preferred_element_type=jnp.float32) \
                + jnp.dot(h1, refs[2 + no + i][...],
                          preferred_element_type=jnp.float32)

    return pl.pallas_call(
        body,
        grid=(m // bn,),
        in_specs=([pl.BlockSpec((bn, 128), lambda i: (i, 0))] * 2
                  + [pl.BlockSpec((128, 128), lambda i: (0, 0))] * 2 * no),
        out_specs=[pl.BlockSpec((bn, 128), lambda i: (i, 0))] * no,
        out_shape=[jax.ShapeDtypeStruct((m, 128), jnp.float32)] * no,
    )(a0p, a1p, *mats0, *mats1)


def _sc_layer(stages, reduce_out=False):
    ins = []
    meta = []
    for n_rows, n_pad, jobs in stages:
        jmeta = []
        for job in jobs:
            jmeta.append((len(ins), job[1].shape[0], job[4]))
            ins.extend(job[:4])
        meta.append((n_rows, jmeta))

    out_type = []
    for n_rows, n_pad, _ in stages:
        shp = (NSUB, HH) if reduce_out else (n_pad, HH)
        out_type.extend([jax.ShapeDtypeStruct(shp, jnp.float32)] * 2)

    mesh = plsc.VectorSubcoreMesh(core_axis_name="core",
                                  subcore_axis_name="subcore")

    @functools.partial(
        pl.kernel,
        out_type=out_type,
        mesh=mesh,
        compiler_params=pltpu.CompilerParams(use_tc_tiling_on_sc=False),
        scratch_types=[
            pltpu.VMEM_SHARED((ACC_ROWS, HH), jnp.float32),
            pltpu.VMEM((ZB, HH), jnp.float32),
            pltpu.VMEM((2, GR * CR, CH), jnp.int32),
            pltpu.VMEM((2, GR * CR, CH), jnp.int32),
            pltpu.VMEM((GR, CR * CH, HH), jnp.float32),
        ] + [pltpu.SemaphoreType.DMA] * (GR + 3),
    )
    def sc_kernel(*refs):
        n_in = len(ins)
        in_refs = refs[:n_in]
        out_refs = refs[n_in:n_in + 2 * len(stages)]
        scr = refs[n_in + 2 * len(stages):]
        acc, zbuf, srcb, dstb, rowb = scr[:5]
        gsem = scr[5:5 + GR]
        ssem = scr[5 + GR]
        isem = scr[6 + GR:8 + GR]

        cid = lax.axis_index("core")
        sid = lax.axis_index("subcore")

        for si, (n_rows, jmeta) in enumerate(meta):
            per = -(-(n_rows // NSUB) // 8) * 8
            last = n_rows - (NSUB - 1) * per
            rbase = sid * per

            @pl.loop(0, ZB)
            def _(i):
                zbuf[i, :] = jnp.zeros((HH,), jnp.float32)

            def zero_slice(m, rbase=rbase):
                nfull, rem = m // ZB, m % ZB
                if nfull:
                    @pl.loop(0, nfull)
                    def _(i):
                        pltpu.sync_copy(
                            zbuf, acc.at[pl.ds(rbase + i * ZB, ZB)])
                if rem:
                    pltpu.sync_copy(
                        zbuf.at[pl.ds(0, rem)],
                        acc.at[pl.ds(rbase + nfull * ZB, rem)])

            @pl.when(sid < NSUB - 1)
            def _():
                zero_slice(per)

            @pl.when(sid == NSUB - 1)
            def _():
                zero_slice(last)

            for (base_i, idx_rows, cached) in jmeta:
                if not cached:
                    continue

                def tload(m, t_ref, rbase=rbase):
                    pltpu.sync_copy(t_ref.at[pl.ds(rbase, m)],
                                    acc.at[pl.ds(TOFF + rbase, m)])

                def tload_core(t_ref):
                    @pl.when(sid < NSUB - 1)
                    def _():
                        tload(per, t_ref)

                    @pl.when(sid == NSUB - 1)
                    def _():
                        tload(last, t_ref)

                @pl.when(cid == 0)
                def _():
                    tload_core(in_refs[base_i + 2])

                @pl.when(cid == 1)
                def _():
                    tload_core(in_refs[base_i + 3])

            plsc.subcore_barrier()

            for (base_i, idx_rows, cached) in jmeta:
                src_ref = in_refs[base_i]
                dst_ref = in_refs[base_i + 1]
                nch = idx_rows // NSUB
                grows = GR * CR
                ngr = nch // grows
                crow0 = sid * nch

                def run_job(table_ref, ngr=ngr, crow0=crow0,
                            src_ref=src_ref, dst_ref=dst_ref):
                    def load_idx(g, b):
                        r = crow0 + g * grows
                        pltpu.async_copy(src_ref.at[pl.ds(r, grows)],
                                         srcb.at[b], isem[b])
                        pltpu.async_copy(dst_ref.at[pl.ds(r, grows)],
                                         dstb.at[b], isem[b])

                    def wait_idx(g, b):
                        r = crow0 + g * grows
                        pltpu.make_async_copy(src_ref.at[pl.ds(r, grows)],
                                              srcb.at[b], isem[b]).wait()
                        pltpu.make_async_copy(dst_ref.at[pl.ds(r, grows)],
                                              dstb.at[b], isem[b]).wait()

                    def do_group(g, b):
                        wait_idx(g, b)

                        @pl.when(g + 1 < ngr)
                        def _():
                            load_idx(g + 1, 1 - b)

                        gcps = [pltpu.async_copy(
                                    table_ref.at[srcb.at[b, k]],
                                    rowb.at[k], gsem[k])
                                for k in range(GR)]
                        scps = []
                        for k in range(GR):
                            gcps[k].wait()
                            scps.append(pltpu.async_copy(
                                rowb.at[k], acc.at[dstb.at[b, k]],
                                ssem, add=True))
                        for cp in scps:
                            cp.wait()

                    load_idx(0, 0)
                    if ngr >= 2:
                        @pl.loop(0, ngr // 2)
                        def _(t):
                            do_group(2 * t, 0)
                            do_group(2 * t + 1, 1)
                    if ngr % 2:
                        do_group(ngr - 1, 0)

                if cached:
                    run_job(acc)
                else:
                    @pl.when(cid == 0)
                    def _():
                        run_job(in_refs[base_i + 2])

                    @pl.when(cid == 1)
                    def _():
                        run_job(in_refs[base_i + 3])

            plsc.subcore_barrier()

            if reduce_out:
                def reduce_slice(m, out_ref, rbase=rbase):
                    nfull, rem = m // ZB, m % ZB
                    rowb[0, 0, :] = jnp.zeros((HH,), jnp.float32)
                    if nfull:
                        @pl.loop(0, nfull)
                        def _(i):
                            pltpu.sync_copy(
                                acc.at[pl.ds(rbase + i * ZB, ZB)], zbuf)
                            rowb[0, 0, :] += lax.fori_loop(
                                0, ZB,
                                lambda j, s: s + jnp.maximum(zbuf[j, :], 0.0),
                                jnp.zeros((HH,), jnp.float32))
                    if rem:
                        pltpu.sync_copy(
                            acc.at[pl.ds(rbase + nfull * ZB, rem)],
                            zbuf.at[pl.ds(0, rem)])
                        rowb[0, 0, :] += lax.fori_loop(
                            0, rem,
                            lambda j, s: s + jnp.maximum(zbuf[j, :], 0.0),
                            jnp.zeros((HH,), jnp.float32))
                    pltpu.sync_copy(rowb.at[0, pl.ds(0, 1)],
                                    out_ref.at[pl.ds(sid, 1)])

                def reduce_core(out_ref):
                    @pl.when(sid < NSUB - 1)
                    def _():
                        reduce_slice(per, out_ref)

                    @pl.when(sid == NSUB - 1)
                    def _():
                        reduce_slice(last, out_ref)

                @pl.when(cid == 0)
                def _():
                    reduce_core(out_refs[2 * si])

                @pl.when(cid == 1)
                def _():
                    reduce_core(out_refs[2 * si + 1])
            else:
                def drain(out_ref, rbase=rbase, per=per, last=last):
                    @pl.when(sid < NSUB - 1)
                    def _():
                        pltpu.sync_copy(acc.at[pl.ds(rbase, per)],
                                        out_ref.at[pl.ds(rbase, per)])

                    @pl.when(sid == NSUB - 1)
                    def _():
                        pltpu.sync_copy(acc.at[pl.ds(rbase, last)],
                                        out_ref.at[pl.ds(rbase, last)])

                @pl.when(cid == 0)
                def _():
                    drain(out_refs[2 * si])

                @pl.when(cid == 1)
                def _():
                    drain(out_refs[2 * si + 1])

            plsc.subcore_barrier()

    return sc_kernel(*ins)


def kernel(x_0, x_1, x_2, adjacency_0, adjacency_1, coadjacency_2,
           incidence_1_rows, incidence_1_cols, incidence_2_rows,
           incidence_2_cols,
           W_in0, b_in0, W_in1, b_in1, W_in2, b_in2,
           W00, W10, W01, W11, W12, W22,
           V00, V01, V11, V12, V22,
           Wo0, bo0, Wo1, bo1, Wo2, bo2):
    t0_mats, t0_bs = _halves(
        jnp.concatenate([_pad_mat(W_in0 @ W00), _pad_mat(W_in0 @ W01)], 1),
        jnp.concatenate([_pad_vec_mat(b_in0, W00), _pad_vec_mat(b_in0, W01)]))
    t1_mats, t1_bs = _halves(
        jnp.concatenate([_pad_mat(W_in1 @ W10), _pad_mat(W_in1 @ W11),
                         _pad_mat(W_in1 @ W12)], 1),
        jnp.concatenate([_pad_vec_mat(b_in1, W10), _pad_vec_mat(b_in1, W11),
                         _pad_vec_mat(b_in1, W12)]))
    t2_mats, t2_bs = _halves(_pad_mat(W_in2 @ W22),
                             _pad_vec_mat(b_in2, W22))

    def mid_mats(u):
        k = u.shape[1]
        m0 = [_kron8(u[:HH, j * HH:(j + 1) * HH]) for j in range(k // HH)]
        m1 = [_kron8(u[HH:, j * HH:(j + 1) * HH]) for j in range(k // HH)]
        return m0, m1

    u0_m0, u0_m1 = mid_mats(
        jnp.concatenate([_pad_mat(V00), _pad_mat(V01)], 1))
    u1_m0, u1_m1 = mid_mats(
        jnp.concatenate([_pad_mat(V11), _pad_mat(V12)], 1))
    u2_m0, u2_m1 = mid_mats(_pad_mat(V22))

    a0_dst = _prep_dst(adjacency_0[0])
    a0_srcT = _prep_src(adjacency_0[1], TOFF)
    a1_dst, a1_src = _prep_dst(adjacency_1[0]), _prep_src(adjacency_1[1])
    c2_dst = _prep_dst(coadjacency_2[0])
    c2_srcT = _prep_src(coadjacency_2[1], TOFF)
    i1r_src = _prep_src(incidence_1_rows)
    i1r_dst = _prep_dst(incidence_1_rows)
    i1c_src = _prep_src(incidence_1_cols)
    i1c_dst = _prep_dst(incidence_1_cols)
    i2r_src = _prep_src(incidence_2_rows)
    i2c_dst = _prep_dst(incidence_2_cols)

    n0p, n1p, n2p = 51200, 102400, 51200
    v = lambda a: a.reshape(a.shape[0] * 8, HH)

    def xpack(x, n, np_):
        xp = x.reshape(n // 8, 8 * HP)
        return jnp.concatenate(
            [xp, jnp.zeros((np_ // 8 - n // 8, 8 * HP), jnp.float32)])

    g00_0, g00_1, g01_0, g01_1 = map(v, _tc_kron_tables(
        xpack(x_0, N0, n0p), t0_mats, t0_bs))
    g10_0, g10_1, g11_0, g11_1, g12_0, g12_1 = map(v, _tc_kron_tables(
        xpack(x_1, N1, n1p), t1_mats, t1_bs))
    g22_0, g22_1 = map(v, _tc_kron_tables(
        xpack(x_2, N2, n2p), t2_mats, t2_bs))

    (acc0_0, acc0_1, acc1_0, acc1_1, acc2_0, acc2_1) = _sc_layer([
        (N0, n0p, [(a0_srcT, a0_dst, g00_0, g00_1, True),
                   (i1c_src, i1r_dst, g10_0, g10_1, False)]),
        (N1, n1p, [(i1r_src, i1c_dst, g01_0, g01_1, False),
                   (a1_src, a1_dst, g11_0, g11_1, False)]),
        (N2, n2p, [(i2r_src, i2c_dst, g12_0, g12_1, False),
                   (c2_srcT, c2_dst, g22_0, g22_1, True)]),
    ])

    p = lambda a: a.reshape(a.shape[0] // 8, 128)
    d00_0, d00_1, d01_0, d01_1 = map(v, _tc_kron_mid(
        p(acc0_0), p(acc0_1), u0_m0, u0_m1))
    d11_0, d11_1, d12_0, d12_1 = map(v, _tc_kron_mid(
        p(acc1_0), p(acc1_1), u1_m0, u1_m1))
    d22_0, d22_1 = map(v, _tc_kron_mid(
        p(acc2_0), p(acc2_1), u2_m0, u2_m1))

    (p0_0, p0_1, p1_0, p1_1, p2_0, p2_1) = _sc_layer([
        (N0, n0p, [(a0_srcT, a0_dst, d00_0, d00_1, True)]),
        (N1, n1p, [(i1r_src, i1c_dst, d01_0, d01_1, False),
                   (a1_src, a1_dst, d11_0, d11_1, False)]),
        (N2, n2p, [(i2r_src, i2c_dst, d12_0, d12_1, False),
                   (c2_srcT, c2_dst, d22_0, d22_1, True)]),
    ], reduce_out=True)

    def head(p_a, p_b, n, wo, bo):
        s = jnp.concatenate([p_a.sum(0), p_b.sum(0)])[None, :]
        wo_p = jnp.zeros((HP, 1), jnp.float32).at[:wo.shape[0]].set(wo)
        return (s @ wo_p)[0] / n + bo

    return (head(p0_0, p0_1, N0, Wo0, bo0) + head(p1_0, p1_1, N1, Wo1, bo1)
            + head(p2_0, p2_1, N2, Wo2, bo2))
